# baseline (speedup 1.0000x reference)
import jax
import jax.numpy as jnp
from jax import lax
from jax.experimental import pallas as pl
from jax.experimental.pallas import tpu as pltpu
from jax.experimental.pallas import tpu_sc as plsc

EMBED = 64
NUM_ROWS = 77
B_TOTAL = 16384 * 200
NUM_WORKERS = 32
PER_WORKER = B_TOTAL // NUM_WORKERS
STEP = 512
NBUF = 2
STEPS = PER_WORKER // STEP
ROW_UNROLL = 16


def _embed_kernel(table_hbm, idx_hbm, out_hbm, tab_v, idx_v, rows_v,
                  osem0, osem1, isem0, isem1):
    cid = lax.axis_index("c")
    sid = lax.axis_index("s")
    wid = sid * 2 + cid
    row_base = wid * PER_WORKER
    osems = [osem0, osem1]
    isems = [isem0, isem1]

    def idx_slice(i):
        return idx_hbm.at[pl.ds(pl.multiple_of(row_base + i * STEP, STEP), STEP)]

    def out_slice(i):
        return out_hbm.at[pl.ds(pl.multiple_of(row_base + i * STEP, STEP), STEP)]

    pltpu.sync_copy(table_hbm, tab_v)

    pltpu.async_copy(idx_slice(0), idx_v.at[0], isems[0])

    @pl.loop(0, STEPS, step=NBUF)
    def _outer(i0):
        for b in range(NBUF):
            i = i0 + b
            nb = (b + 1) % NBUF

            pltpu.make_async_copy(idx_slice(0), idx_v.at[b], isems[b]).wait()

            @pl.when(i + 1 < STEPS)
            def _prefetch():
                pltpu.async_copy(idx_slice(i + 1), idx_v.at[nb], isems[nb])

            @pl.when(i0 >= NBUF)
            def _reclaim():
                pltpu.make_async_copy(
                    rows_v.at[b], out_slice(0), osems[b]
                ).wait()

            @pl.loop(0, STEP, step=ROW_UNROLL)
            def _rows(r0):
                iv = idx_v[b, pl.ds(r0, ROW_UNROLL)]
                for u in range(ROW_UNROLL):
                    r = r0 + u
                    ir = iv[u]
                    for c in range(EMBED // 16):
                        rows_v[b, r, pl.ds(c * 16, 16)] = (
                            tab_v[ir, pl.ds(c * 16, 16)]
                        )

            pltpu.async_copy(rows_v.at[b], out_slice(i), osems[b])

    for b in range(NBUF):
        pltpu.make_async_copy(rows_v.at[b], out_slice(0), osems[b]).wait()


@jax.jit
def kernel(day, table):
    idx1d = day.reshape(B_TOTAL).astype(jnp.int32)
    mesh = plsc.VectorSubcoreMesh(core_axis_name="c", subcore_axis_name="s")
    out = pl.kernel(
        _embed_kernel,
        mesh=mesh,
        compiler_params=pltpu.CompilerParams(use_tc_tiling_on_sc=False),
        out_type=jax.ShapeDtypeStruct((B_TOTAL, EMBED), jnp.float32),
        scratch_types=[
            pltpu.VMEM((NUM_ROWS, EMBED), jnp.float32),
            pltpu.VMEM((NBUF, STEP), jnp.int32),
            pltpu.VMEM((NBUF, STEP, EMBED), jnp.float32),
            pltpu.SemaphoreType.DMA,
            pltpu.SemaphoreType.DMA,
            pltpu.SemaphoreType.DMA,
            pltpu.SemaphoreType.DMA,
        ],
    )(table, idx1d)
    return out.reshape(day.shape[0], day.shape[1], EMBED)
